# Initial kernel scaffold; baseline (speedup 1.0000x reference)
import jax
import jax.numpy as jnp
from jax.experimental import pallas as pl


def kernel(x, h, edge_index, W_val, b_val, W_sign, b_sign, W_que, b_que, W_ih, b_ih, W_hh, b_hh):
    raise NotImplementedError("write your pallas kernel here")


# speedup vs baseline: 6.1702x; 6.1702x over previous
import functools

import jax
import jax.numpy as jnp
from jax import lax
from jax.experimental import pallas as pl
from jax.experimental.pallas import tpu as pltpu
from jax.experimental.pallas import tpu_sc as plsc

N = 10000
E = 320000
HID = 128
MSG = 32
KEY = 32

NC = 2
NS = 16
NW = NC * NS
EPW = E // NW
CH = 80
NCHUNK = EPW // CH
DEN_R, DEN_C = 128, 80


def _edge_body(src_hbm, dst_hbm, s_hbm, q_hbm, v_hbm,
               acc_out, den_out,
               idx_s, idx_d, srow, qrow, vrow,
               den_loc, iota128, zer, zden, acc_s, den_s,
               sem0, sem1, sem2):
    cid = lax.axis_index("c")
    sid = lax.axis_index("s")
    wid = cid * NS + sid

    lanes = lax.iota(jnp.int32, 16)
    zf = jnp.zeros((16,), jnp.float32)

    def _z1(i, _):
        zer[i // 2, pl.ds((i % 2) * 16, 16)] = zf
        return 0
    lax.fori_loop(0, 400, _z1, 0)

    def _z2(i, _):
        zden[i // 5, pl.ds((i % 5) * 16, 16)] = zf
        return 0
    lax.fori_loop(0, 40, _z2, 0)

    def _z3(i, _):
        den_loc[i // 5, pl.ds((i % 5) * 16, 16)] = zf
        return 0
    lax.fori_loop(0, DEN_R * 5, _z3, 0)

    def _z4(i, _):
        iota128[pl.ds(i * 16, 16)] = lanes + i * 16
        return 0
    lax.fori_loop(0, 8, _z4, 0)

    @pl.when(sid < 10)
    def _():
        def _z5(j, _):
            pltpu.sync_copy(zer, acc_s.at[pl.ds(sid * 1000 + j * 200, 200), :])
            return 0
        lax.fori_loop(0, 5, _z5, 0)
    pltpu.sync_copy(zden, den_s.at[pl.ds(sid * 8, 8), :])
    plsc.subcore_barrier()

    ebase = wid * EPW

    def _chunk(j, _):
        base = ebase + j * CH
        pltpu.sync_copy(src_hbm.at[pl.ds(base, CH)], idx_s)
        pltpu.sync_copy(dst_hbm.at[pl.ds(base, CH)], idx_d)
        c1 = pltpu.async_copy(s_hbm.at[idx_s], srow, sem0)
        c2 = pltpu.async_copy(q_hbm.at[idx_d], qrow, sem1)
        c3 = pltpu.async_copy(v_hbm.at[idx_s], vrow, sem2)
        c1.wait()
        c2.wait()
        c3.wait()

        def _grp(g, _):
            rows = g * 16 + lanes
            dv = idx_d[pl.ds(g * 16, 16)]

            def _dot(k, e):
                kk = jnp.full((16,), k, jnp.int32)
                sv = plsc.load_gather(srow, [rows, kk])
                qv = plsc.load_gather(qrow, [rows, kk])
                return e + sv * qv
            e = lax.fori_loop(0, MSG, _dot, jnp.zeros((16,), jnp.float32))
            w = jnp.exp(e)
            plsc.addupdate_scatter(den_loc, [dv // DEN_C, dv % DEN_C], w)

            def _wv(k, _):
                kk = jnp.full((16,), k, jnp.int32)
                vv = plsc.load_gather(vrow, [rows, kk])
                plsc.store_scatter(vrow, [rows, kk], vv * w)
                return 0
            lax.fori_loop(0, MSG, _wv, 0)
            return 0
        lax.fori_loop(0, 5, _grp, 0)
        pltpu.sync_copy(vrow, acc_s.at[idx_d], add=True)
        return 0
    lax.fori_loop(0, NCHUNK, _chunk, 0)

    pltpu.sync_copy(den_loc, den_s.at[iota128], add=True)
    plsc.subcore_barrier()

    @pl.when(sid < 10)
    def _():
        pltpu.sync_copy(acc_s.at[pl.ds(sid * 1000, 1000), :],
                        acc_out.at[cid, pl.ds(sid * 1000, 1000), :])

    @pl.when(sid < 8)
    def _():
        pltpu.sync_copy(den_s.at[pl.ds(sid * 16, 16), :],
                        den_out.at[cid, pl.ds(sid * 16, 16), :])


_edge_call = functools.partial(
    pl.kernel,
    mesh=plsc.VectorSubcoreMesh(core_axis_name="c", subcore_axis_name="s"),
    compiler_params=pltpu.CompilerParams(
        needs_layout_passes=False, use_tc_tiling_on_sc=False),
    out_type=[
        jax.ShapeDtypeStruct((NC, N, MSG), jnp.float32),
        jax.ShapeDtypeStruct((NC, DEN_R, DEN_C), jnp.float32),
    ],
    scratch_types=[
        pltpu.VMEM((CH,), jnp.int32),
        pltpu.VMEM((CH,), jnp.int32),
        pltpu.VMEM((CH, KEY), jnp.float32),
        pltpu.VMEM((CH, KEY), jnp.float32),
        pltpu.VMEM((CH, MSG), jnp.float32),
        pltpu.VMEM((DEN_R, DEN_C), jnp.float32),
        pltpu.VMEM((128,), jnp.int32),
        pltpu.VMEM((200, MSG), jnp.float32),
        pltpu.VMEM((8, DEN_C), jnp.float32),
        pltpu.VMEM_SHARED((N, MSG), jnp.float32),
        pltpu.VMEM_SHARED((DEN_R, DEN_C), jnp.float32),
        pltpu.SemaphoreType.DMA,
        pltpu.SemaphoreType.DMA,
        pltpu.SemaphoreType.DMA,
    ],
)(_edge_body)


RB = 400
GRID = N // RB


def _proj_body(x_ref, h_ref, w_ref, b_ref, s_ref, q_ref, v_ref):
    inp = jnp.concatenate([x_ref[...], h_ref[...]], axis=1)
    out = lax.dot_general(inp, w_ref[...], (((1,), (1,)), ((), ())),
                          preferred_element_type=jnp.float32) + b_ref[...]
    s_ref[...] = out[:, 0:KEY] * (1.0 / KEY)
    q_ref[...] = out[:, KEY:2 * KEY]
    v_ref[...] = out[:, 2 * KEY:2 * KEY + MSG]


def _proj(x, h, W_cat, b_cat):
    return pl.pallas_call(
        _proj_body,
        grid=(GRID,),
        in_specs=[
            pl.BlockSpec((RB, HID), lambda i: (i, 0)),
            pl.BlockSpec((RB, HID), lambda i: (i, 0)),
            pl.BlockSpec((2 * KEY + MSG, 2 * HID), lambda i: (0, 0)),
            pl.BlockSpec((1, 2 * KEY + MSG), lambda i: (0, 0)),
        ],
        out_specs=[
            pl.BlockSpec((RB, KEY), lambda i: (i, 0)),
            pl.BlockSpec((RB, KEY), lambda i: (i, 0)),
            pl.BlockSpec((RB, MSG), lambda i: (i, 0)),
        ],
        out_shape=[
            jax.ShapeDtypeStruct((N, KEY), jnp.float32),
            jax.ShapeDtypeStruct((N, KEY), jnp.float32),
            jax.ShapeDtypeStruct((N, MSG), jnp.float32),
        ],
    )(x, h, W_cat, b_cat)


def _gru_body(with_proj, x_ref, h_ref, a0_ref, a1_ref, d0_ref, d1_ref,
              wih_ref, bih_ref, whh_ref, bhh_ref, wc_ref, bc_ref,
              h_out, *proj_outs):
    x = x_ref[...]
    hv = h_ref[...]
    den = d0_ref[...] + d1_ref[...]
    inv = jnp.where(den > 0.0, 1.0 / jnp.where(den > 0.0, den, 1.0), 0.0)
    c = (a0_ref[...] + a1_ref[...]) * inv
    inp = jnp.concatenate([x, c], axis=1)
    gi = lax.dot_general(inp, wih_ref[...], (((1,), (1,)), ((), ())),
                         preferred_element_type=jnp.float32) + bih_ref[...]
    gh = lax.dot_general(hv, whh_ref[...], (((1,), (1,)), ((), ())),
                         preferred_element_type=jnp.float32) + bhh_ref[...]
    r = jax.nn.sigmoid(gi[:, 0:HID] + gh[:, 0:HID])
    z = jax.nn.sigmoid(gi[:, HID:2 * HID] + gh[:, HID:2 * HID])
    n = jnp.tanh(gi[:, 2 * HID:] + r * gh[:, 2 * HID:])
    hn = (1.0 - z) * n + z * hv
    h_out[...] = hn
    if with_proj:
        s_ref, q_ref, v_ref = proj_outs
        inp2 = jnp.concatenate([x, hn], axis=1)
        out = lax.dot_general(inp2, wc_ref[...], (((1,), (1,)), ((), ())),
                              preferred_element_type=jnp.float32) + bc_ref[...]
        s_ref[...] = out[:, 0:KEY] * (1.0 / KEY)
        q_ref[...] = out[:, KEY:2 * KEY]
        v_ref[...] = out[:, 2 * KEY:2 * KEY + MSG]


def _gru(x, h, a0, a1, d0, d1, W_ih, b_ih, W_hh, b_hh, W_cat, b_cat, with_proj):
    out_specs = [pl.BlockSpec((RB, HID), lambda i: (i, 0))]
    out_shape = [jax.ShapeDtypeStruct((N, HID), jnp.float32)]
    if with_proj:
        out_specs += [
            pl.BlockSpec((RB, KEY), lambda i: (i, 0)),
            pl.BlockSpec((RB, KEY), lambda i: (i, 0)),
            pl.BlockSpec((RB, MSG), lambda i: (i, 0)),
        ]
        out_shape += [
            jax.ShapeDtypeStruct((N, KEY), jnp.float32),
            jax.ShapeDtypeStruct((N, KEY), jnp.float32),
            jax.ShapeDtypeStruct((N, MSG), jnp.float32),
        ]
    return pl.pallas_call(
        functools.partial(_gru_body, with_proj),
        grid=(GRID,),
        in_specs=[
            pl.BlockSpec((RB, HID), lambda i: (i, 0)),
            pl.BlockSpec((RB, HID), lambda i: (i, 0)),
            pl.BlockSpec((RB, MSG), lambda i: (i, 0)),
            pl.BlockSpec((RB, MSG), lambda i: (i, 0)),
            pl.BlockSpec((RB, 1), lambda i: (i, 0)),
            pl.BlockSpec((RB, 1), lambda i: (i, 0)),
            pl.BlockSpec((3 * HID, HID + MSG), lambda i: (0, 0)),
            pl.BlockSpec((1, 3 * HID), lambda i: (0, 0)),
            pl.BlockSpec((3 * HID, HID), lambda i: (0, 0)),
            pl.BlockSpec((1, 3 * HID), lambda i: (0, 0)),
            pl.BlockSpec((2 * KEY + MSG, 2 * HID), lambda i: (0, 0)),
            pl.BlockSpec((1, 2 * KEY + MSG), lambda i: (0, 0)),
        ],
        out_specs=out_specs,
        out_shape=out_shape,
    )(x, h, a0, a1, d0, d1, W_ih, b_ih, W_hh, b_hh, W_cat, b_cat)


def kernel(x, h, edge_index, W_val, b_val, W_sign, b_sign, W_que, b_que,
           W_ih, b_ih, W_hh, b_hh):
    src = edge_index[0].astype(jnp.int32)
    dst = edge_index[1].astype(jnp.int32)
    W_cat = jnp.concatenate([W_sign, W_que, W_val], axis=0)
    b_cat = jnp.concatenate([b_sign, b_que, b_val], axis=0)[None, :]
    b_ih2 = b_ih[None, :]
    b_hh2 = b_hh[None, :]

    s_t, q_t, v_t = _proj(x, h, W_cat, b_cat)
    acc, den = _edge_call(src, dst, s_t, q_t, v_t)
    d = den.reshape(NC, DEN_R * DEN_C)[:, :N]
    h1, s_t, q_t, v_t = _gru(x, h, acc[0], acc[1],
                             d[0][:, None], d[1][:, None],
                             W_ih, b_ih2, W_hh, b_hh2, W_cat, b_cat, True)
    acc, den = _edge_call(src, dst, s_t, q_t, v_t)
    d = den.reshape(NC, DEN_R * DEN_C)[:, :N]
    (h2,) = _gru(x, h1, acc[0], acc[1],
                 d[0][:, None], d[1][:, None],
                 W_ih, b_ih2, W_hh, b_hh2, W_cat, b_cat, False)
    return h2


# speedup vs baseline: 9.4619x; 1.5335x over previous
import functools

import jax
import jax.numpy as jnp
from jax import lax
from jax.experimental import pallas as pl
from jax.experimental.pallas import tpu as pltpu
from jax.experimental.pallas import tpu_sc as plsc

N = 10000
E = 320000
HID = 128
MSG = 32
KEY = 32

NC = 2
NS = 16
NW = NC * NS
EPW = E // NW
CH = 80
NCHUNK = EPW // CH
DEN_R, DEN_C = 128, 80


def _edge_body(src_hbm, dst_hbm, s_hbm, q_hbm, v_hbm,
               acc_out, den_out,
               src_loc, dst_loc, srow0, qrow0, vrow0, srow1, qrow1, vrow1,
               den_loc, iota128, zer, zden, acc_s, den_s,
               sem_s0, sem_q0, sem_v0, sem_s1, sem_q1, sem_v1):
    cid = lax.axis_index("c")
    sid = lax.axis_index("s")
    wid = cid * NS + sid

    lanes = lax.iota(jnp.int32, 16)
    zf = jnp.zeros((16,), jnp.float32)

    bufs = ((srow0, qrow0, vrow0, sem_s0, sem_q0, sem_v0),
            (srow1, qrow1, vrow1, sem_s1, sem_q1, sem_v1))

    def _z1(i, _):
        zer[i // 2, pl.ds((i % 2) * 16, 16)] = zf
        return 0
    lax.fori_loop(0, 400, _z1, 0)

    def _z2(i, _):
        zden[i // 5, pl.ds((i % 5) * 16, 16)] = zf
        return 0
    lax.fori_loop(0, 40, _z2, 0)

    def _z3(i, _):
        den_loc[i // 5, pl.ds((i % 5) * 16, 16)] = zf
        return 0
    lax.fori_loop(0, DEN_R * 5, _z3, 0)

    def _z4(i, _):
        iota128[pl.ds(i * 16, 16)] = lanes + i * 16
        return 0
    lax.fori_loop(0, 8, _z4, 0)

    pltpu.sync_copy(src_hbm.at[wid], src_loc)
    pltpu.sync_copy(dst_hbm.at[wid], dst_loc)

    @pl.when(sid < 10)
    def _():
        def _z5(j, _):
            pltpu.sync_copy(zer, acc_s.at[pl.ds(sid * 1000 + j * 200, 200), :])
            return 0
        lax.fori_loop(0, 5, _z5, 0)
    pltpu.sync_copy(zden, den_s.at[pl.ds(sid * 8, 8), :])
    plsc.subcore_barrier()

    def _start(j, b):
        srow, qrow, vrow, ss, sq, sv = bufs[b]
        pltpu.async_copy(s_hbm.at[src_loc.at[j]], srow, ss)
        pltpu.async_copy(q_hbm.at[dst_loc.at[j]], qrow, sq)
        pltpu.async_copy(v_hbm.at[src_loc.at[j]], vrow, sv)

    def _wait(b):
        srow, qrow, vrow, ss, sq, sv = bufs[b]
        pltpu.make_async_copy(s_hbm.at[src_loc.at[0]], srow, ss).wait()
        pltpu.make_async_copy(q_hbm.at[dst_loc.at[0]], qrow, sq).wait()
        pltpu.make_async_copy(v_hbm.at[src_loc.at[0]], vrow, sv).wait()

    def _compute(j, b):
        srow, qrow, vrow, *_ = bufs[b]

        def _grp(g, _):
            rows = g * 16 + lanes
            dv = dst_loc[j, pl.ds(g * 16, 16)]

            @plsc.parallel_loop(0, MSG, unroll=4,
                                carry=jnp.zeros((16,), jnp.float32))
            def e(k, acc):
                kk = jnp.full((16,), k, jnp.int32)
                sv = plsc.load_gather(srow, [rows, kk])
                qv = plsc.load_gather(qrow, [rows, kk])
                return acc + sv * qv
            w = jnp.exp(e)
            plsc.addupdate_scatter(den_loc, [dv // DEN_C, dv % DEN_C], w)

            @plsc.parallel_loop(0, MSG, unroll=4)
            def _(k):
                kk = jnp.full((16,), k, jnp.int32)
                vv = plsc.load_gather(vrow, [rows, kk])
                plsc.store_scatter(vrow, [rows, kk], vv * w)
            return 0
        lax.fori_loop(0, 5, _grp, 0)
        pltpu.sync_copy(vrow, acc_s.at[dst_loc.at[j]], add=True)

    _start(0, 0)

    def _pair(jj, _):
        j0 = 2 * jj
        _start(j0 + 1, 1)
        _wait(0)
        _compute(j0, 0)
        _start(j0 + 2, 0)
        _wait(1)
        _compute(j0 + 1, 1)
        return 0
    lax.fori_loop(0, (NCHUNK - 1) // 2, _pair, 0)
    _wait(0)
    _compute(NCHUNK - 1, 0)

    pltpu.sync_copy(den_loc, den_s.at[iota128], add=True)
    plsc.subcore_barrier()

    @pl.when(sid < 10)
    def _():
        pltpu.sync_copy(acc_s.at[pl.ds(sid * 1000, 1000), :],
                        acc_out.at[cid, pl.ds(sid * 1000, 1000), :])

    @pl.when(sid < 8)
    def _():
        pltpu.sync_copy(den_s.at[pl.ds(sid * 16, 16), :],
                        den_out.at[cid, pl.ds(sid * 16, 16), :])


_edge_call = functools.partial(
    pl.kernel,
    mesh=plsc.VectorSubcoreMesh(core_axis_name="c", subcore_axis_name="s"),
    compiler_params=pltpu.CompilerParams(
        needs_layout_passes=False, use_tc_tiling_on_sc=False),
    out_type=[
        jax.ShapeDtypeStruct((NC, N, MSG), jnp.float32),
        jax.ShapeDtypeStruct((NC, DEN_R, DEN_C), jnp.float32),
    ],
    scratch_types=[
        pltpu.VMEM((NCHUNK, CH), jnp.int32),
        pltpu.VMEM((NCHUNK, CH), jnp.int32),
        pltpu.VMEM((CH, KEY), jnp.float32),
        pltpu.VMEM((CH, KEY), jnp.float32),
        pltpu.VMEM((CH, MSG), jnp.float32),
        pltpu.VMEM((CH, KEY), jnp.float32),
        pltpu.VMEM((CH, KEY), jnp.float32),
        pltpu.VMEM((CH, MSG), jnp.float32),
        pltpu.VMEM((DEN_R, DEN_C), jnp.float32),
        pltpu.VMEM((128,), jnp.int32),
        pltpu.VMEM((200, MSG), jnp.float32),
        pltpu.VMEM((8, DEN_C), jnp.float32),
        pltpu.VMEM_SHARED((N, MSG), jnp.float32),
        pltpu.VMEM_SHARED((DEN_R, DEN_C), jnp.float32),
        pltpu.SemaphoreType.DMA,
        pltpu.SemaphoreType.DMA,
        pltpu.SemaphoreType.DMA,
        pltpu.SemaphoreType.DMA,
        pltpu.SemaphoreType.DMA,
        pltpu.SemaphoreType.DMA,
    ],
)(_edge_body)


RB = 400
GRID = N // RB


def _proj_body(x_ref, h_ref, w_ref, b_ref, s_ref, q_ref, v_ref):
    inp = jnp.concatenate([x_ref[...], h_ref[...]], axis=1)
    out = lax.dot_general(inp, w_ref[...], (((1,), (1,)), ((), ())),
                          preferred_element_type=jnp.float32) + b_ref[...]
    s_ref[...] = out[:, 0:KEY] * (1.0 / KEY)
    q_ref[...] = out[:, KEY:2 * KEY]
    v_ref[...] = out[:, 2 * KEY:2 * KEY + MSG]


def _proj(x, h, W_cat, b_cat):
    return pl.pallas_call(
        _proj_body,
        grid=(GRID,),
        in_specs=[
            pl.BlockSpec((RB, HID), lambda i: (i, 0)),
            pl.BlockSpec((RB, HID), lambda i: (i, 0)),
            pl.BlockSpec((2 * KEY + MSG, 2 * HID), lambda i: (0, 0)),
            pl.BlockSpec((1, 2 * KEY + MSG), lambda i: (0, 0)),
        ],
        out_specs=[
            pl.BlockSpec((RB, KEY), lambda i: (i, 0)),
            pl.BlockSpec((RB, KEY), lambda i: (i, 0)),
            pl.BlockSpec((RB, MSG), lambda i: (i, 0)),
        ],
        out_shape=[
            jax.ShapeDtypeStruct((N, KEY), jnp.float32),
            jax.ShapeDtypeStruct((N, KEY), jnp.float32),
            jax.ShapeDtypeStruct((N, MSG), jnp.float32),
        ],
    )(x, h, W_cat, b_cat)


def _gru_body(with_proj, x_ref, h_ref, a0_ref, a1_ref, d0_ref, d1_ref,
              wih_ref, bih_ref, whh_ref, bhh_ref, wc_ref, bc_ref,
              h_out, *proj_outs):
    x = x_ref[...]
    hv = h_ref[...]
    den = d0_ref[...] + d1_ref[...]
    inv = jnp.where(den > 0.0, 1.0 / jnp.where(den > 0.0, den, 1.0), 0.0)
    c = (a0_ref[...] + a1_ref[...]) * inv
    inp = jnp.concatenate([x, c], axis=1)
    gi = lax.dot_general(inp, wih_ref[...], (((1,), (1,)), ((), ())),
                         preferred_element_type=jnp.float32) + bih_ref[...]
    gh = lax.dot_general(hv, whh_ref[...], (((1,), (1,)), ((), ())),
                         preferred_element_type=jnp.float32) + bhh_ref[...]
    r = jax.nn.sigmoid(gi[:, 0:HID] + gh[:, 0:HID])
    z = jax.nn.sigmoid(gi[:, HID:2 * HID] + gh[:, HID:2 * HID])
    n = jnp.tanh(gi[:, 2 * HID:] + r * gh[:, 2 * HID:])
    hn = (1.0 - z) * n + z * hv
    h_out[...] = hn
    if with_proj:
        s_ref, q_ref, v_ref = proj_outs
        inp2 = jnp.concatenate([x, hn], axis=1)
        out = lax.dot_general(inp2, wc_ref[...], (((1,), (1,)), ((), ())),
                              preferred_element_type=jnp.float32) + bc_ref[...]
        s_ref[...] = out[:, 0:KEY] * (1.0 / KEY)
        q_ref[...] = out[:, KEY:2 * KEY]
        v_ref[...] = out[:, 2 * KEY:2 * KEY + MSG]


def _gru(x, h, a0, a1, d0, d1, W_ih, b_ih, W_hh, b_hh, W_cat, b_cat, with_proj):
    out_specs = [pl.BlockSpec((RB, HID), lambda i: (i, 0))]
    out_shape = [jax.ShapeDtypeStruct((N, HID), jnp.float32)]
    if with_proj:
        out_specs += [
            pl.BlockSpec((RB, KEY), lambda i: (i, 0)),
            pl.BlockSpec((RB, KEY), lambda i: (i, 0)),
            pl.BlockSpec((RB, MSG), lambda i: (i, 0)),
        ]
        out_shape += [
            jax.ShapeDtypeStruct((N, KEY), jnp.float32),
            jax.ShapeDtypeStruct((N, KEY), jnp.float32),
            jax.ShapeDtypeStruct((N, MSG), jnp.float32),
        ]
    return pl.pallas_call(
        functools.partial(_gru_body, with_proj),
        grid=(GRID,),
        in_specs=[
            pl.BlockSpec((RB, HID), lambda i: (i, 0)),
            pl.BlockSpec((RB, HID), lambda i: (i, 0)),
            pl.BlockSpec((RB, MSG), lambda i: (i, 0)),
            pl.BlockSpec((RB, MSG), lambda i: (i, 0)),
            pl.BlockSpec((RB, 1), lambda i: (i, 0)),
            pl.BlockSpec((RB, 1), lambda i: (i, 0)),
            pl.BlockSpec((3 * HID, HID + MSG), lambda i: (0, 0)),
            pl.BlockSpec((1, 3 * HID), lambda i: (0, 0)),
            pl.BlockSpec((3 * HID, HID), lambda i: (0, 0)),
            pl.BlockSpec((1, 3 * HID), lambda i: (0, 0)),
            pl.BlockSpec((2 * KEY + MSG, 2 * HID), lambda i: (0, 0)),
            pl.BlockSpec((1, 2 * KEY + MSG), lambda i: (0, 0)),
        ],
        out_specs=out_specs,
        out_shape=out_shape,
    )(x, h, a0, a1, d0, d1, W_ih, b_ih, W_hh, b_hh, W_cat, b_cat)


def kernel(x, h, edge_index, W_val, b_val, W_sign, b_sign, W_que, b_que,
           W_ih, b_ih, W_hh, b_hh):
    src = edge_index[0].astype(jnp.int32).reshape(NW, NCHUNK, CH)
    dst = edge_index[1].astype(jnp.int32).reshape(NW, NCHUNK, CH)
    W_cat = jnp.concatenate([W_sign, W_que, W_val], axis=0)
    b_cat = jnp.concatenate([b_sign, b_que, b_val], axis=0)[None, :]
    b_ih2 = b_ih[None, :]
    b_hh2 = b_hh[None, :]

    s_t, q_t, v_t = _proj(x, h, W_cat, b_cat)
    acc, den = _edge_call(src, dst, s_t, q_t, v_t)
    d = den.reshape(NC, DEN_R * DEN_C)[:, :N]
    h1, s_t, q_t, v_t = _gru(x, h, acc[0], acc[1],
                             d[0][:, None], d[1][:, None],
                             W_ih, b_ih2, W_hh, b_hh2, W_cat, b_cat, True)
    acc, den = _edge_call(src, dst, s_t, q_t, v_t)
    d = den.reshape(NC, DEN_R * DEN_C)[:, :N]
    (h2,) = _gru(x, h1, acc[0], acc[1],
                 d[0][:, None], d[1][:, None],
                 W_ih, b_ih2, W_hh, b_hh2, W_cat, b_cat, False)
    return h2
